# baseline (speedup 1.0000x reference)
import jax
import jax.numpy as jnp
from jax import lax
from jax.experimental import pallas as pl
from jax.experimental.pallas import tpu as pltpu
from jax.experimental.pallas import tpu_sc as plsc

B = 16
T = 20
K = 16
DIMS = (4096, 2048, 1024, 512)
PREV = (8192, 4096, 2048, 1024)
ALPHA = 0.9
BETA = 0.85
NQ = 8
NR = 4
NP = NR // 2
NG_PER_CORE = 2
L = 16
ILV = plsc.PackFormat.INTERLEAVED
WCH = (512, 256, 128, 128)
NSTAGE = T + 4


def _body(input_h,
          knnP_all, wT_all,
          thr0_h, bias0_h, thr1_h, bias1_h,
          thr2_h, bias2_h, thr3_h, bias3_h,
          mo0, mo1, mo2, mo3, so0, so1, so2, so3,
          x0a, x0b,
          xp1a, xp1b, xp2a, xp2b, xp3a, xp3b,
          kv0, wv0, tv0, bv0, sy0, me0, sp0,
          kv1, wv1, tv1, bv1, sy1, me1, sp1,
          kv2, wv2, tv2, bv2, sy2, me2, sp2,
          kv3, wv3, tv3, bv3, sy3, me3, sp3,
          spp00, spp01, spp10, spp11, spp20, spp21,
          xcA0, xcA1, xcA2, xcB0, xcB1, xcB2,
          msem0, msem1, msem2, msem3, ssem0, ssem1, ssem2, ssem3,
          sem_a, sem_b, xsem, psem):
    c = lax.axis_index("c")
    s_id = lax.axis_index("s")
    gl = s_id // NQ
    q = s_id % NQ
    b0 = (c * NG_PER_CORE + gl) * NR

    thr_h = (thr0_h, thr1_h, thr2_h, thr3_h)
    bias_h = (bias0_h, bias1_h, bias2_h, bias3_h)
    mo = (mo0, mo1, mo2, mo3)
    so = (so0, so1, so2, so3)
    kv = (kv0, kv1, kv2, kv3)
    wv = (wv0, wv1, wv2, wv3)
    tv = (tv0, tv1, tv2, tv3)
    bv = (bv0, bv1, bv2, bv3)
    sy = (sy0, sy1, sy2, sy3)
    me = (me0, me1, me2, me3)
    sp = (sp0, sp1, sp2, sp3)
    spp = ((spp00, spp01), (spp10, spp11), (spp20, spp21))
    xcA = (xcA0, xcA1, xcA2)
    xcB = (xcB0, xcB1, xcB2)
    xpl = {1: (xp1a, xp1b), 2: (xp2a, xp2b), 3: (xp3a, xp3b)}
    msem = (msem0, msem1, msem2, msem3)
    ssem = (ssem0, ssem1, ssem2, ssem3)

    zeros16 = jnp.zeros((L,), jnp.float32)
    roffv = tuple(jnp.full((L,), r * PREV[0], jnp.int32)
                  for r in range(1, NR))

    def fetch_x0(t, buf, sem):
        for r in range(NR):
            pltpu.async_copy(input_h.at[b0 + r, t],
                             buf.at[pl.ds(r * PREV[0], PREV[0])], sem)

    def wait_x0(buf, sem):
        for r in range(NR):
            pltpu.make_async_copy(input_h.at[b0, 0],
                                  buf.at[pl.ds(r * PREV[0], PREV[0])],
                                  sem).wait()

    def out_slice(o, i):
        W = WCH[i]
        return o.at[0, pl.ds(b0, NR), pl.ds(q * W, W)]

    def fire_reads(xc):
        for i in (1, 2, 3):
            d = PREV[i]
            for p in range(NP):
                pltpu.async_copy(xc[i - 1].at[gl * NP + p, :],
                                 xpl[i][p].at[pl.ds(0, d)], xsem)

    def wait_reads(xc):
        for i in (1, 2, 3):
            d = PREV[i]
            for p in range(NP):
                pltpu.make_async_copy(xc[i - 1].at[gl * NP + p, :],
                                      xpl[i][p].at[pl.ds(0, d)],
                                      xsem).wait()

    for i in range(4):
        W = WCH[i]

        loff = sum(DIMS[:i])

        def prolog(i=i, W=W, loff=loff):
            j0 = q * W
            jt = loff + j0
            pltpu.sync_copy(knnP_all.at[:, pl.ds(jt, W)], kv[i])
            pltpu.sync_copy(wT_all.at[:, pl.ds(jt, W)], wv[i])
            pltpu.sync_copy(thr_h[i].at[pl.ds(j0, W)], tv[i])
            pltpu.sync_copy(bias_h[i].at[pl.ds(j0, W)], bv[i])

            def zbody(jb, carry):
                o = pl.multiple_of(jb * L, L)
                for r in range(NR):
                    sy[i][r, pl.ds(o, L)] = zeros16
                    me[i][r, pl.ds(o, L)] = zeros16
                return carry
            lax.fori_loop(0, W // L, zbody, None)
            pltpu.async_copy(me[i], out_slice(mo[i], i), msem[i])
            pltpu.async_copy(sp[i], out_slice(so[i], i), ssem[i])
        if i == 3:
            pl.when(q < 4)(prolog)
        else:
            prolog()
    fetch_x0(0, x0a, sem_a)
    fire_reads(xcB)

    def do_layer(i, t, x0buf):
        W = WCH[i]
        kvi, wvi, tvi, bvi = kv[i], wv[i], tv[i], bv[i]
        syi, mei, spi = sy[i], me[i], sp[i]

        pltpu.make_async_copy(mei, out_slice(mo[i], i), msem[i]).wait()
        pltpu.make_async_copy(spi, out_slice(so[i], i), ssem[i]).wait()

        def jbody(o):
            o = pl.multiple_of(o, L)
            bb = bvi[pl.ds(o, L)]
            acc = [bb] * NR
            for kp in range(K // 2):
                ab = plsc.bitcast(kvi[kp, pl.ds(o, L)], jnp.int16)
                ia, ib = plsc.unpack(ab, format=ILV)
                for k, idx in ((2 * kp, ia), (2 * kp + 1, ib)):
                    wk = wvi[k, pl.ds(o, L)]
                    if i == 0:
                        g = [plsc.load_gather(x0buf, [idx])]
                        g += [plsc.load_gather(x0buf, [idx + roffv[r - 1]])
                              for r in range(1, NR)]
                    else:
                        g = []
                        for p in range(NP):
                            gp = plsc.load_gather(xpl[i][p], [idx])
                            gb = plsc.bitcast(gp, jnp.bfloat16)
                            g += list(plsc.unpack(gb, format=ILV))
                    for r in range(NR):
                        acc[r] = acc[r] + g[r] * wk
            thrv = tvi[pl.ds(o, L)]
            spks = []
            for r in range(NR):
                m = mei[r, pl.ds(o, L)]
                sn = ALPHA * syi[r, pl.ds(o, L)] + acc[r]
                mn = BETA * m + sn - jnp.where(m > thrv, thrv, 0.0)
                spkv = jnp.where(mn > thrv, 1.0, 0.0)
                syi[r, pl.ds(o, L)] = sn
                mei[r, pl.ds(o, L)] = mn
                spi[r, pl.ds(o, L)] = spkv
                spks.append(spkv)
            if i < 3:
                for p in range(NP):
                    pk = plsc.pack(spks[2 * p], spks[2 * p + 1], format=ILV)
                    spp[i][p][pl.ds(o, L)] = plsc.bitcast(pk, jnp.int32)
        plsc.parallel_loop(0, W, step=L, unroll=4)(jbody)

        pltpu.async_copy(mei, mo[i].at[t, pl.ds(b0, NR), pl.ds(q * W, W)],
                         msem[i])
        pltpu.async_copy(spi, so[i].at[t, pl.ds(b0, NR), pl.ds(q * W, W)],
                         ssem[i])

    def stage(s, x0buf, xc, xc_prev):
        pl.when(s < T)(lambda: do_layer(0, s, x0buf))
        wait_reads(xc_prev)
        for i in (1, 2):
            pl.when((s >= i) & (s < T + i))(
                lambda i=i: do_layer(i, s - i, x0buf))
        pl.when((s >= 3) & (s < T + 3) & (q < 4))(
            lambda: do_layer(3, s - 3, x0buf))
        for i in range(3):
            W = WCH[i]
            for p in range(NP):
                pltpu.async_copy(spp[i][p],
                                 xc[i].at[gl * NP + p, pl.ds(q * W, W)],
                                 psem)
        for i in range(3):
            W = WCH[i]
            for p in range(NP):
                pltpu.make_async_copy(spp[i][p],
                                      xc[i].at[gl * NP + p,
                                               pl.ds(q * W, W)],
                                      psem).wait()
        plsc.subcore_barrier()
        fire_reads(xc)

    def pair(pidx, carry):
        s0 = pidx * 2
        fetch_x0(jnp.minimum(s0 + 1, T - 1), x0b, sem_b)
        wait_x0(x0a, sem_a)
        stage(s0, x0a, xcA, xcB)
        fetch_x0(jnp.minimum(s0 + 2, T - 1), x0a, sem_a)
        wait_x0(x0b, sem_b)
        stage(s0 + 1, x0b, xcB, xcA)
        return carry

    lax.fori_loop(0, NSTAGE // 2, pair, None)

    wait_x0(x0a, sem_a)
    wait_reads(xcB)
    for i in range(4):
        def drain(i=i):
            pltpu.make_async_copy(me[i], out_slice(mo[i], i),
                                  msem[i]).wait()
            pltpu.make_async_copy(sp[i], out_slice(so[i], i),
                                  ssem[i]).wait()
        if i == 3:
            pl.when(q < 4)(drain)
        else:
            drain()


@jax.jit
def _run(input, knnT, wT, thr, bias):
    mesh = plsc.VectorSubcoreMesh(core_axis_name="c", subcore_axis_name="s")
    out_type = (
        tuple(jax.ShapeDtypeStruct((T, B, d), jnp.float32) for d in DIMS)
        + tuple(jax.ShapeDtypeStruct((T, B, d), jnp.float32) for d in DIMS)
    )
    scratch = [
        pltpu.VMEM((NR * PREV[0],), jnp.float32),
        pltpu.VMEM((NR * PREV[0],), jnp.float32),
    ]
    for i in (1, 2, 3):
        scratch += [pltpu.VMEM((PREV[i],), jnp.int32)] * NP
    for W in WCH:
        scratch += [
            pltpu.VMEM((K // 2, W), jnp.int32),
            pltpu.VMEM((K, W), jnp.float32),
            pltpu.VMEM((W,), jnp.float32),
            pltpu.VMEM((W,), jnp.float32),
            pltpu.VMEM((NR, W), jnp.float32),
            pltpu.VMEM((NR, W), jnp.float32),
            pltpu.VMEM((NR, W), jnp.float32),
        ]
    for W in WCH[:3]:
        scratch += [pltpu.VMEM((W,), jnp.int32)] * NP
    scratch += [pltpu.VMEM_SHARED((NG_PER_CORE * NP, d), jnp.int32)
                for d in DIMS[:3]] * 2
    scratch += [pltpu.SemaphoreType.DMA] * 12
    flat_in = [input, knnT, wT]
    for i in range(4):
        flat_in += [thr[i], bias[i]]
    run = pl.kernel(_body, out_type=out_type, mesh=mesh,
                    scratch_types=scratch,
                    compiler_params=pltpu.CompilerParams(
                        needs_layout_passes=False))
    outs = run(*flat_in)
    return outs[:4], outs[4:]


def kernel(input, weight0, bias0, knn0, thr0, weight1, bias1, knn1, thr1,
           weight2, bias2, knn2, thr2, weight3, bias3, knn3, thr3,
           fc_w, fc_b):
    kA = jnp.concatenate([knn0, knn1, knn2, knn3], axis=0).astype(jnp.int32)
    knnT = (kA[:, 0::2] | (kA[:, 1::2] << 16)).T
    wT = jnp.concatenate([weight0, weight1, weight2, weight3], axis=0).T
    thr = (thr0, thr1, thr2, thr3)
    bias = tuple(b.reshape(-1) for b in (bias0, bias1, bias2, bias3))
    mem_rec, spk_rec = _run(input, knnT, wT, thr, bias)
    angles = jnp.dot(mem_rec[3][T - 1], fc_w.T) + fc_b
    return tuple(mem_rec) + tuple(spk_rec) + (angles,)


# speedup vs baseline: 1.0537x; 1.0537x over previous
import jax
import jax.numpy as jnp
from jax import lax
from jax.experimental import pallas as pl
from jax.experimental.pallas import tpu as pltpu
from jax.experimental.pallas import tpu_sc as plsc

B = 16
T = 20
K = 16
DIMS = (4096, 2048, 1024, 512)
PREV = (8192, 4096, 2048, 1024)
ALPHA = 0.9
BETA = 0.85
NQ = 8
NR = 4
NP = NR // 2
NG_PER_CORE = 2
L = 16
ILV = plsc.PackFormat.INTERLEAVED
WCH = (512, 256, 128, 128)
NSTAGE = T + 4


def _body(input_h,
          knnP_all, wT_all,
          thr0_h, bias0_h, thr1_h, bias1_h,
          thr2_h, bias2_h, thr3_h, bias3_h,
          mo0, mo1, mo2, mo3, so0, so1, so2, so3,
          x0a, x0b,
          xp1a, xp1b, xp2a, xp2b, xp3a, xp3b,
          kv0, wv0, tv0, bv0, sy0, me0, sp0,
          kv1, wv1, tv1, bv1, sy1, me1, sp1,
          kv2, wv2, tv2, bv2, sy2, me2, sp2,
          kv3, wv3, tv3, bv3, sy3, me3, sp3,
          spp00, spp01, spp10, spp11, spp20, spp21,
          xcA0, xcA1, xcA2, xcB0, xcB1, xcB2,
          msem0, msem1, msem2, msem3, ssem0, ssem1, ssem2, ssem3,
          sem_a, sem_b, xsem, psem):
    c = lax.axis_index("c")
    s_id = lax.axis_index("s")
    gl = s_id // NQ
    q = s_id % NQ
    b0 = (c * NG_PER_CORE + gl) * NR

    thr_h = (thr0_h, thr1_h, thr2_h, thr3_h)
    bias_h = (bias0_h, bias1_h, bias2_h, bias3_h)
    mo = (mo0, mo1, mo2, mo3)
    so = (so0, so1, so2, so3)
    kv = (kv0, kv1, kv2, kv3)
    wv = (wv0, wv1, wv2, wv3)
    tv = (tv0, tv1, tv2, tv3)
    bv = (bv0, bv1, bv2, bv3)
    sy = (sy0, sy1, sy2, sy3)
    me = (me0, me1, me2, me3)
    sp = (sp0, sp1, sp2, sp3)
    spp = ((spp00, spp01), (spp10, spp11), (spp20, spp21))
    xcA = (xcA0, xcA1, xcA2)
    xcB = (xcB0, xcB1, xcB2)
    xpl = {1: (xp1a, xp1b), 2: (xp2a, xp2b), 3: (xp3a, xp3b)}
    msem = (msem0, msem1, msem2, msem3)
    ssem = (ssem0, ssem1, ssem2, ssem3)

    zeros16 = jnp.zeros((L,), jnp.float32)
    roffv = tuple(jnp.full((L,), r * PREV[0], jnp.int32)
                  for r in range(1, NR))

    def fetch_x0(t, buf, sem):
        for r in range(NR):
            pltpu.async_copy(input_h.at[b0 + r, t],
                             buf.at[pl.ds(r * PREV[0], PREV[0])], sem)

    def wait_x0(buf, sem):
        for r in range(NR):
            pltpu.make_async_copy(input_h.at[b0, 0],
                                  buf.at[pl.ds(r * PREV[0], PREV[0])],
                                  sem).wait()

    def out_slice(o, i):
        W = WCH[i]
        return o.at[0, pl.ds(b0, NR), pl.ds(q * W, W)]

    def fire_reads(xc):
        for i in (1, 2, 3):
            d = PREV[i]
            for p in range(NP):
                pltpu.async_copy(xc[i - 1].at[gl * NP + p, :],
                                 xpl[i][p].at[pl.ds(0, d)], xsem)

    def wait_reads(xc):
        for i in (1, 2, 3):
            d = PREV[i]
            for p in range(NP):
                pltpu.make_async_copy(xc[i - 1].at[gl * NP + p, :],
                                      xpl[i][p].at[pl.ds(0, d)],
                                      xsem).wait()

    for i in range(4):
        W = WCH[i]

        loff = sum(DIMS[:i])

        def prolog(i=i, W=W, loff=loff):
            j0 = q * W
            jt = loff + j0
            pltpu.sync_copy(knnP_all.at[:, pl.ds(jt, W)], kv[i])
            pltpu.sync_copy(wT_all.at[:, pl.ds(jt, W)], wv[i])
            pltpu.sync_copy(thr_h[i].at[pl.ds(j0, W)], tv[i])
            pltpu.sync_copy(bias_h[i].at[pl.ds(j0, W)], bv[i])

            def zbody(jb, carry):
                o = pl.multiple_of(jb * L, L)
                for r in range(NR):
                    sy[i][r, pl.ds(o, L)] = zeros16
                    me[i][r, pl.ds(o, L)] = zeros16
                return carry
            lax.fori_loop(0, W // L, zbody, None)
            pltpu.async_copy(me[i], out_slice(mo[i], i), msem[i])
            pltpu.async_copy(sp[i], out_slice(so[i], i), ssem[i])
        if i == 3:
            pl.when(q < 4)(prolog)
        else:
            prolog()
    fetch_x0(0, x0a, sem_a)
    fire_reads(xcB)

    def do_layer(i, t, x0buf):
        W = WCH[i]
        kvi, wvi, tvi, bvi = kv[i], wv[i], tv[i], bv[i]
        syi, mei, spi = sy[i], me[i], sp[i]

        pltpu.make_async_copy(mei, out_slice(mo[i], i), msem[i]).wait()
        pltpu.make_async_copy(spi, out_slice(so[i], i), ssem[i]).wait()

        def jbody(o):
            o = pl.multiple_of(o, L)
            bb = bvi[pl.ds(o, L)]
            acc = [bb] * NR
            for kp in range(K // 2):
                ab = plsc.bitcast(kvi[kp, pl.ds(o, L)], jnp.int16)
                ia, ib = plsc.unpack(ab, format=ILV)
                for k, idx in ((2 * kp, ia), (2 * kp + 1, ib)):
                    wk = wvi[k, pl.ds(o, L)]
                    if i == 0:
                        g = [plsc.load_gather(x0buf, [idx])]
                        g += [plsc.load_gather(x0buf, [idx + roffv[r - 1]])
                              for r in range(1, NR)]
                    else:
                        g = []
                        for p in range(NP):
                            gp = plsc.load_gather(xpl[i][p], [idx])
                            gb = plsc.bitcast(gp, jnp.bfloat16)
                            g += list(plsc.unpack(gb, format=ILV))
                    for r in range(NR):
                        acc[r] = acc[r] + g[r] * wk
            thrv = tvi[pl.ds(o, L)]
            spks = []
            for r in range(NR):
                m = mei[r, pl.ds(o, L)]
                sn = ALPHA * syi[r, pl.ds(o, L)] + acc[r]
                mn = BETA * m + sn - jnp.where(m > thrv, thrv, 0.0)
                spkv = jnp.where(mn > thrv, 1.0, 0.0)
                syi[r, pl.ds(o, L)] = sn
                mei[r, pl.ds(o, L)] = mn
                spi[r, pl.ds(o, L)] = spkv
                spks.append(spkv)
            if i < 3:
                for p in range(NP):
                    pk = plsc.pack(spks[2 * p], spks[2 * p + 1], format=ILV)
                    spp[i][p][pl.ds(o, L)] = plsc.bitcast(pk, jnp.int32)
        plsc.parallel_loop(0, W, step=L, unroll=2)(jbody)

        pltpu.async_copy(mei, mo[i].at[t, pl.ds(b0, NR), pl.ds(q * W, W)],
                         msem[i])
        pltpu.async_copy(spi, so[i].at[t, pl.ds(b0, NR), pl.ds(q * W, W)],
                         ssem[i])

    def stage(s, x0buf, xc, xc_prev):
        pl.when(s < T)(lambda: do_layer(0, s, x0buf))
        wait_reads(xc_prev)
        for i in (1, 2):
            pl.when((s >= i) & (s < T + i))(
                lambda i=i: do_layer(i, s - i, x0buf))
        pl.when((s >= 3) & (s < T + 3) & (q < 4))(
            lambda: do_layer(3, s - 3, x0buf))
        for i in range(3):
            W = WCH[i]
            for p in range(NP):
                pltpu.async_copy(spp[i][p],
                                 xc[i].at[gl * NP + p, pl.ds(q * W, W)],
                                 psem)
        for i in range(3):
            W = WCH[i]
            for p in range(NP):
                pltpu.make_async_copy(spp[i][p],
                                      xc[i].at[gl * NP + p,
                                               pl.ds(q * W, W)],
                                      psem).wait()
        plsc.subcore_barrier()
        fire_reads(xc)

    def pair(pidx, carry):
        s0 = pidx * 2
        fetch_x0(jnp.minimum(s0 + 1, T - 1), x0b, sem_b)
        wait_x0(x0a, sem_a)
        stage(s0, x0a, xcA, xcB)
        fetch_x0(jnp.minimum(s0 + 2, T - 1), x0a, sem_a)
        wait_x0(x0b, sem_b)
        stage(s0 + 1, x0b, xcB, xcA)
        return carry

    lax.fori_loop(0, NSTAGE // 2, pair, None)

    wait_x0(x0a, sem_a)
    wait_reads(xcB)
    for i in range(4):
        def drain(i=i):
            pltpu.make_async_copy(me[i], out_slice(mo[i], i),
                                  msem[i]).wait()
            pltpu.make_async_copy(sp[i], out_slice(so[i], i),
                                  ssem[i]).wait()
        if i == 3:
            pl.when(q < 4)(drain)
        else:
            drain()


@jax.jit
def _run(input, knnT, wT, thr, bias):
    mesh = plsc.VectorSubcoreMesh(core_axis_name="c", subcore_axis_name="s")
    out_type = (
        tuple(jax.ShapeDtypeStruct((T, B, d), jnp.float32) for d in DIMS)
        + tuple(jax.ShapeDtypeStruct((T, B, d), jnp.float32) for d in DIMS)
    )
    scratch = [
        pltpu.VMEM((NR * PREV[0],), jnp.float32),
        pltpu.VMEM((NR * PREV[0],), jnp.float32),
    ]
    for i in (1, 2, 3):
        scratch += [pltpu.VMEM((PREV[i],), jnp.int32)] * NP
    for W in WCH:
        scratch += [
            pltpu.VMEM((K // 2, W), jnp.int32),
            pltpu.VMEM((K, W), jnp.float32),
            pltpu.VMEM((W,), jnp.float32),
            pltpu.VMEM((W,), jnp.float32),
            pltpu.VMEM((NR, W), jnp.float32),
            pltpu.VMEM((NR, W), jnp.float32),
            pltpu.VMEM((NR, W), jnp.float32),
        ]
    for W in WCH[:3]:
        scratch += [pltpu.VMEM((W,), jnp.int32)] * NP
    scratch += [pltpu.VMEM_SHARED((NG_PER_CORE * NP, d), jnp.int32)
                for d in DIMS[:3]] * 2
    scratch += [pltpu.SemaphoreType.DMA] * 12
    flat_in = [input, knnT, wT]
    for i in range(4):
        flat_in += [thr[i], bias[i]]
    run = pl.kernel(_body, out_type=out_type, mesh=mesh,
                    scratch_types=scratch,
                    compiler_params=pltpu.CompilerParams(
                        needs_layout_passes=False))
    outs = run(*flat_in)
    return outs[:4], outs[4:]


def kernel(input, weight0, bias0, knn0, thr0, weight1, bias1, knn1, thr1,
           weight2, bias2, knn2, thr2, weight3, bias3, knn3, thr3,
           fc_w, fc_b):
    kA = jnp.concatenate([knn0, knn1, knn2, knn3], axis=0).astype(jnp.int32)
    knnT = (kA[:, 0::2] | (kA[:, 1::2] << 16)).T
    wT = jnp.concatenate([weight0, weight1, weight2, weight3], axis=0).T
    thr = (thr0, thr1, thr2, thr3)
    bias = tuple(b.reshape(-1) for b in (bias0, bias1, bias2, bias3))
    mem_rec, spk_rec = _run(input, knnT, wT, thr, bias)
    angles = jnp.dot(mem_rec[3][T - 1], fc_w.T) + fc_b
    return tuple(mem_rec) + tuple(spk_rec) + (angles,)
